# baseline (speedup 1.0000x reference)
import functools

import jax
import jax.numpy as jnp
from jax import lax
from jax.experimental import pallas as pl
from jax.experimental.pallas import tpu as pltpu
from jax.experimental.pallas import tpu_sc as plsc

N = 50000
D = 256
EPS = 1e-12
NC, NS, L = 2, 16, 16
NW = NC * NS
C = 16
NCHUNKS = N // C


def _vsqrt(x):
    i = plsc.bitcast(x, jnp.int32)
    i = jnp.int32(0x5F3759DF) - (i >> 1)
    y = plsc.bitcast(i, jnp.float32)
    for _ in range(3):
        y = y * (1.5 - (0.5 * x * y) * y)
    return x * y


def _pair_loss(na, nb, dab):
    sa = jnp.maximum(_vsqrt(na), EPS)
    sb = jnp.maximum(_vsqrt(nb), EPS)
    return na / (sa * sa) + nb / (sb * sb) - 2.0 * (dab / (sa * sb))


_mesh = plsc.VectorSubcoreMesh(core_axis_name="c", subcore_axis_name="s")


@functools.partial(
    pl.kernel,
    mesh=_mesh,
    out_type=jax.ShapeDtypeStruct((NW, L), jnp.float32),
    scratch_types=[pltpu.VMEM((C, D + 1), jnp.float32) for _ in range(4)]
    + [pltpu.VMEM((L,), jnp.float32)],
    compiler_params=pltpu.CompilerParams(
        use_tc_tiling_on_sc=True, needs_layout_passes=False
    ),
)
def _sc_loss(p1, q2, p2, q1, out, b1, b2, b3, b4, lbuf):
    wid = lax.axis_index("s") * NC + lax.axis_index("c")
    nmine = (NCHUNKS - wid + NW - 1) // NW
    row_iota = lax.iota(jnp.int32, L)
    zero = jnp.zeros((L,), jnp.float32)

    def chunk_body(i, loss):
        base = (wid + i * NW) * C
        pltpu.sync_copy(p1.at[pl.ds(base, C)], b1.at[:, :D])
        pltpu.sync_copy(q2.at[pl.ds(base, C)], b2.at[:, :D])
        pltpu.sync_copy(p2.at[pl.ds(base, C)], b3.at[:, :D])
        pltpu.sync_copy(q1.at[pl.ds(base, C)], b4.at[:, :D])

        def d_body(d, accs):
            na1, nb1, dd1, na2, nb2, dd2 = accs
            col = lax.full((L,), d, jnp.int32)
            a = plsc.load_gather(b1, [row_iota, col])
            b = plsc.load_gather(b2, [row_iota, col])
            c = plsc.load_gather(b3, [row_iota, col])
            e = plsc.load_gather(b4, [row_iota, col])
            return (
                na1 + a * a,
                nb1 + b * b,
                dd1 + a * b,
                na2 + c * c,
                nb2 + e * e,
                dd2 + c * e,
            )

        na1, nb1, dd1, na2, nb2, dd2 = lax.fori_loop(
            0, D, d_body, (zero, zero, zero, zero, zero, zero), unroll=4
        )
        return loss + _pair_loss(na1, nb1, dd1) + _pair_loss(na2, nb2, dd2)

    lbuf[...] = lax.fori_loop(0, nmine, chunk_body, zero)
    pltpu.sync_copy(lbuf, out.at[wid])


@jax.jit
def kernel(pred1, proj2, pred2, proj1):
    partials = _sc_loss(pred1, proj2, pred2, proj1)
    return jnp.sum(partials) / 2.0


# speedup vs baseline: 3.3724x; 3.3724x over previous
import functools

import jax
import jax.numpy as jnp
from jax import lax
from jax.experimental import pallas as pl
from jax.experimental.pallas import tpu as pltpu
from jax.experimental.pallas import tpu_sc as plsc

N = 50000
D = 256
EPS = 1e-12
NC, NS, L = 2, 16, 16
NW = NC * NS
C = 16
NCHUNKS = N // C
NITER = (NCHUNKS + NW - 1) // NW


def _vsqrt(x):
    i = plsc.bitcast(x, jnp.int32)
    i = jnp.int32(0x5F3759DF) - (i >> 1)
    y = plsc.bitcast(i, jnp.float32)
    for _ in range(3):
        y = y * (1.5 - (0.5 * x * y) * y)
    return x * y


def _pair_loss(na, nb, dab):
    sa = jnp.maximum(_vsqrt(na), EPS)
    sb = jnp.maximum(_vsqrt(nb), EPS)
    return na / (sa * sa) + nb / (sb * sb) - 2.0 * (dab / (sa * sb))


_mesh = plsc.VectorSubcoreMesh(core_axis_name="c", subcore_axis_name="s")


@functools.partial(
    pl.kernel,
    mesh=_mesh,
    out_type=jax.ShapeDtypeStruct((NW, L), jnp.float32),
    scratch_types=[pltpu.VMEM((C, D + 1), jnp.float32) for _ in range(8)]
    + [
        pltpu.VMEM((L,), jnp.float32),
        pltpu.SemaphoreType.DMA,
        pltpu.SemaphoreType.DMA,
    ],
    compiler_params=pltpu.CompilerParams(
        use_tc_tiling_on_sc=False, needs_layout_passes=False
    ),
)
def _sc_loss(p1, q2, p2, q1, out, b0, b1, b2, b3, b4, b5, b6, b7, lbuf, s0, s1):
    wid = lax.axis_index("s") * NC + lax.axis_index("c")
    row_iota = lax.iota(jnp.int32, L)
    zero = jnp.zeros((L,), jnp.float32)
    srcs = (p1, q2, p2, q1)
    bufs = ((b0, b1, b2, b3), (b4, b5, b6, b7))
    sems = (s0, s1)

    def chunk_of(i):
        return wid + i * NW

    def start(i, slot):
        @pl.when(chunk_of(i) < NCHUNKS)
        def _():
            base = chunk_of(i) * C
            for src, dst in zip(srcs, bufs[slot]):
                pltpu.async_copy(src.at[pl.ds(base, C)], dst.at[:, :D], sems[slot])

    def wait(i, slot):
        @pl.when(chunk_of(i) < NCHUNKS)
        def _():
            base = chunk_of(i) * C
            for src, dst in zip(srcs, bufs[slot]):
                pltpu.make_async_copy(
                    src.at[pl.ds(base, C)], dst.at[:, :D], sems[slot]
                ).wait()

    def compute(slot, loss):
        c1, c2, c3, c4 = bufs[slot]

        def d_body(d, accs):
            na1, nb1, dd1, na2, nb2, dd2 = accs
            col = lax.full((L,), d, jnp.int32)
            a = plsc.load_gather(c1, [row_iota, col])
            b = plsc.load_gather(c2, [row_iota, col])
            c = plsc.load_gather(c3, [row_iota, col])
            e = plsc.load_gather(c4, [row_iota, col])
            return (
                na1 + a * a,
                nb1 + b * b,
                dd1 + a * b,
                na2 + c * c,
                nb2 + e * e,
                dd2 + c * e,
            )

        na1, nb1, dd1, na2, nb2, dd2 = lax.fori_loop(
            0, D, d_body, (zero, zero, zero, zero, zero, zero), unroll=8
        )
        return loss + _pair_loss(na1, nb1, dd1) + _pair_loss(na2, nb2, dd2)

    start(0, 0)

    def body(j, loss):
        for k in (0, 1):
            i = 2 * j + k
            start(i + 1, 1 - k)
            wait(i, k)
            loss = jnp.where(chunk_of(i) < NCHUNKS, compute(k, loss), loss)
        return loss

    lbuf[...] = lax.fori_loop(0, NITER // 2, body, zero)
    pltpu.sync_copy(lbuf, out.at[wid])


@jax.jit
def kernel(pred1, proj2, pred2, proj1):
    partials = _sc_loss(pred1, proj2, pred2, proj1)
    return jnp.sum(partials) / 2.0
